# baseline (speedup 1.0000x reference)
import functools

import jax
import jax.numpy as jnp
from jax import lax
from jax.experimental import pallas as pl
from jax.experimental.pallas import tpu as pltpu
from jax.experimental.pallas import tpu_sc as plsc

D = 32
DP = 128
S = 100
SH = 50
CB = 16
RPC = SH * CB
TJB = 8


@functools.lru_cache(maxsize=None)
def _make_gather(BT, V):
    info = plsc.get_sparse_core_info()
    NC, NS, L = info.num_cores, info.num_subcores, info.num_lanes
    NW = NC * NS
    assert L == CB and S % SH == 0
    assert BT % (NW * CB) == 0
    b_per_w = BT // NW
    n_chunks = (b_per_w // CB) * (S // SH)
    s_blocks = S // SH
    mesh = plsc.VectorSubcoreMesh(core_axis_name="c", subcore_axis_name="s")

    @functools.partial(
        pl.kernel,
        mesh=mesh,
        compiler_params=pltpu.CompilerParams(
            use_tc_tiling_on_sc=False, needs_layout_passes=False),
        out_type=jax.ShapeDtypeStruct((S, D // 8, BT // 128, 8, 128),
                                      jnp.float32),
        scratch_types=[
            pltpu.VMEM((2, SH, CB), jnp.int32),
            pltpu.VMEM((2, RPC), jnp.int32),
            pltpu.VMEM((2, RPC, D), jnp.float32),
            pltpu.VMEM((2, SH, D // 8, 8, CB), jnp.float32),
            pltpu.SemaphoreType.DMA,
            pltpu.SemaphoreType.DMA,
            pltpu.SemaphoreType.DMA,
            pltpu.SemaphoreType.DMA,
        ],
    )
    def gather_kernel(idx_hbm, table_hbm, out_hbm, idxb, fidx, rows, outb,
                      sem_g0, sem_g1, sem_o0, sem_o1):
        wid = lax.axis_index("s") * NC + lax.axis_index("c")
        b_base = wid * b_per_w
        iota = lax.iota(jnp.int32, L)
        ti_lo = lax.div(iota, 8)
        ti_hi = ti_lo + 2
        rl = lax.rem(iota, 8)

        def chunk_coords(i):
            s0 = lax.rem(i, s_blocks) * SH
            b0 = b_base + lax.div(i, s_blocks) * CB
            return s0, b0

        def stage_and_fire(i, p):
            s0, b0 = chunk_coords(i)
            pltpu.sync_copy(
                idx_hbm.at[pl.ds(s0, SH), pl.ds(b0, CB)], idxb.at[p])

            def flat_body(s, carry):
                fidx[p, pl.ds(s * CB, CB)] = idxb[p, s, :]
                return carry

            lax.fori_loop(0, SH, flat_body, 0)
            pltpu.async_copy(table_hbm.at[fidx.at[p]],
                             rows.at[p], sem_g0 if p == 0 else sem_g1)

        def wait_gather(p):
            pltpu.make_async_copy(
                table_hbm.at[pl.ds(0, RPC)], rows.at[p],
                sem_g0 if p == 0 else sem_g1).wait()

        def out_slice(s0, tj, cl):
            return out_hbm.at[pl.ds(s0, SH), :, tj, :, pl.ds(cl, CB)]

        def wait_out(p):
            pltpu.make_async_copy(
                outb.at[p], out_slice(0, 0, b_base % 128),
                sem_o0 if p == 0 else sem_o1).wait()

        def transpose_and_fire(i, p):
            def s_body(s, carry):
                s_vec = jnp.full((L,), 0, jnp.int32) + s
                for b in range(0, CB, 2):
                    r0 = s * CB + b
                    v = [rows[p, r0, pl.ds(0, L)],
                         rows[p, r0, pl.ds(L, L)],
                         rows[p, r0 + 1, pl.ds(0, L)],
                         rows[p, r0 + 1, pl.ds(L, L)]]
                    b_vec = jnp.full((L,), b, jnp.int32)
                    b_vec1 = jnp.full((L,), b + 1, jnp.int32)
                    plsc.store_scatter(
                        outb.at[p], [s_vec, ti_lo, rl, b_vec], v[0])
                    plsc.store_scatter(
                        outb.at[p], [s_vec, ti_hi, rl, b_vec], v[1])
                    plsc.store_scatter(
                        outb.at[p], [s_vec, ti_lo, rl, b_vec1], v[2])
                    plsc.store_scatter(
                        outb.at[p], [s_vec, ti_hi, rl, b_vec1], v[3])
                return carry

            lax.fori_loop(0, SH, s_body, 0)
            s0, b0 = chunk_coords(i)
            pltpu.async_copy(
                outb.at[p],
                out_slice(s0, lax.div(b0, 128), lax.rem(b0, 128)),
                sem_o0 if p == 0 else sem_o1)

        stage_and_fire(0, 0)

        def body(i, carry):
            p = lax.rem(i, 2)

            @pl.when(i + 1 < n_chunks)
            def _():
                @pl.when(p == 0)
                def _():
                    stage_and_fire(i + 1, 1)

                @pl.when(p == 1)
                def _():
                    stage_and_fire(i + 1, 0)

            @pl.when(p == 0)
            def _():
                wait_gather(0)

                @pl.when(i >= 2)
                def _():
                    wait_out(0)

                transpose_and_fire(i, 0)

            @pl.when(p == 1)
            def _():
                wait_gather(1)

                @pl.when(i >= 2)
                def _():
                    wait_out(1)

                transpose_and_fire(i, 1)

            return carry

        lax.fori_loop(0, n_chunks, body, 0)
        wait_out(0)
        wait_out(1)

    return gather_kernel


@functools.lru_cache(maxsize=None)
def _make_detile(NTJ):
    info = plsc.get_sparse_core_info()
    NC, NS, L = info.num_cores, info.num_subcores, info.num_lanes
    NW = NC * NS
    base_n, extra = NTJ // NW, NTJ % NW
    mesh = plsc.VectorSubcoreMesh(core_axis_name="c", subcore_axis_name="s")

    @functools.partial(
        pl.kernel,
        mesh=mesh,
        compiler_params=pltpu.CompilerParams(
            use_tc_tiling_on_sc=False, needs_layout_passes=False),
        out_type=jax.ShapeDtypeStruct((NTJ * 128, D), jnp.float32),
        scratch_types=[
            pltpu.VMEM((2, D // 8, TJB, 8, 128), jnp.float32),
            pltpu.VMEM((2, TJB * 128, D), jnp.float32),
            pltpu.SemaphoreType.DMA,
            pltpu.SemaphoreType.DMA,
            pltpu.SemaphoreType.DMA,
            pltpu.SemaphoreType.DMA,
        ],
    )
    def detile_kernel(w5_hbm, table_hbm, sbuf, tbuf, si0, si1, so0, so1):
        wid = lax.axis_index("s") * NC + lax.axis_index("c")
        start = wid * base_n + lax.min(wid, extra)
        count = base_n + jnp.where(wid < extra, 1, 0)
        n_blk = lax.div(count + TJB - 1, TJB)
        iota = lax.iota(jnp.int32, L)

        def blk_start(j):
            return start + lax.min(j * TJB, count - TJB)

        def stage(j, p):
            pltpu.async_copy(w5_hbm.at[:, pl.ds(blk_start(j), TJB), :, :],
                             sbuf.at[p], si0 if p == 0 else si1)

        def wait_stage(p):
            pltpu.make_async_copy(w5_hbm.at[:, pl.ds(0, TJB), :, :],
                                  sbuf.at[p], si0 if p == 0 else si1).wait()

        def wait_wb(p):
            pltpu.make_async_copy(tbuf.at[p],
                                  table_hbm.at[pl.ds(0, TJB * 128)],
                                  so0 if p == 0 else so1).wait()

        def transpose_fire(j, p):
            def c_body(tl, carry):
                def c_inner(c0, carry2):
                    c_vec = iota + (tl * 128 + c0 * L)
                    for d in range(D):
                        v = sbuf[p, d // 8, tl, d % 8, pl.ds(c0 * L, L)]
                        plsc.store_scatter(
                            tbuf.at[p],
                            [c_vec, jnp.full((L,), d, jnp.int32)], v)
                    return carry2

                lax.fori_loop(0, 128 // L, c_inner, 0)
                return carry

            lax.fori_loop(0, TJB, c_body, 0)
            pltpu.async_copy(
                tbuf.at[p], table_hbm.at[pl.ds(blk_start(j) * 128, TJB * 128)],
                so0 if p == 0 else so1)

        stage(0, 0)

        def body(j, carry):
            p = lax.rem(j, 2)

            @pl.when(j + 1 < n_blk)
            def _():
                @pl.when(p == 0)
                def _():
                    stage(j + 1, 1)

                @pl.when(p == 1)
                def _():
                    stage(j + 1, 0)

            @pl.when(p == 0)
            def _():
                wait_stage(0)

                @pl.when(j >= 2)
                def _():
                    wait_wb(0)

                transpose_fire(j, 0)

            @pl.when(p == 1)
            def _():
                wait_stage(1)

                @pl.when(j >= 2)
                def _():
                    wait_wb(1)

                transpose_fire(j, 1)

            return carry

        lax.fori_loop(0, n_blk, body, 0)
        wait_wb(0)
        wait_wb(1)

    return detile_kernel


def kernel(input, weight):
    idx_t = jnp.transpose(input).astype(jnp.int32)
    V = weight.shape[0]
    vpad = (-V) % 128
    wpad = jnp.pad(weight, ((0, vpad), (0, 0)))
    ntj = (V + vpad) // 128
    w4 = jnp.reshape(jnp.transpose(wpad), (D // 8, 8, ntj, 128))
    w5 = jnp.transpose(w4, (0, 2, 1, 3))
    table = _make_detile(ntj)(w5)
    out5 = _make_gather(idx_t.shape[1], V + vpad)(idx_t, table)
    BT = idx_t.shape[1]
    out = jnp.transpose(out5, (2, 4, 0, 1, 3))
    return jnp.reshape(out, (BT, S, D))


# speedup vs baseline: 1.1941x; 1.1941x over previous
import functools

import jax
import jax.numpy as jnp
from jax import lax
from jax.experimental import pallas as pl
from jax.experimental.pallas import tpu as pltpu
from jax.experimental.pallas import tpu_sc as plsc

D = 32
DP = 128
S = 100
SH = 50
CB = 16
RPC = SH * CB
TJB = 6


@functools.lru_cache(maxsize=None)
def _make_gather(BT, V):
    info = plsc.get_sparse_core_info()
    NC, NS, L = info.num_cores, info.num_subcores, info.num_lanes
    NW = NC * NS
    assert L == CB and S % SH == 0
    assert BT % (NW * CB) == 0
    b_per_w = BT // NW
    n_chunks = (b_per_w // CB) * (S // SH)
    s_blocks = S // SH
    mesh = plsc.VectorSubcoreMesh(core_axis_name="c", subcore_axis_name="s")

    @functools.partial(
        pl.kernel,
        mesh=mesh,
        compiler_params=pltpu.CompilerParams(
            use_tc_tiling_on_sc=False, needs_layout_passes=False),
        out_type=jax.ShapeDtypeStruct((S, D // 8, BT // 128, 8, 128),
                                      jnp.float32),
        scratch_types=[
            pltpu.VMEM((2, SH, CB), jnp.int32),
            pltpu.VMEM((2, RPC), jnp.int32),
            pltpu.VMEM((2, RPC, D), jnp.float32),
            pltpu.VMEM((2, SH, D // 8, 8, CB), jnp.float32),
            pltpu.SemaphoreType.DMA,
            pltpu.SemaphoreType.DMA,
            pltpu.SemaphoreType.DMA,
            pltpu.SemaphoreType.DMA,
        ],
    )
    def gather_kernel(idx_hbm, table_hbm, out_hbm, idxb, fidx, rows, outb,
                      sem_g0, sem_g1, sem_o0, sem_o1):
        wid = lax.axis_index("s") * NC + lax.axis_index("c")
        b_base = wid * b_per_w
        iota = lax.iota(jnp.int32, L)
        ti_lo = lax.div(iota, 8)
        ti_hi = ti_lo + 2
        rl = lax.rem(iota, 8)

        def chunk_coords(i):
            s0 = lax.rem(i, s_blocks) * SH
            b0 = b_base + lax.div(i, s_blocks) * CB
            return s0, b0

        def stage_and_fire(i, p):
            s0, b0 = chunk_coords(i)
            pltpu.sync_copy(
                idx_hbm.at[pl.ds(s0, SH), pl.ds(b0, CB)], idxb.at[p])

            def flat_body(s, carry):
                fidx[p, pl.ds(s * CB, CB)] = idxb[p, s, :]
                return carry

            lax.fori_loop(0, SH, flat_body, 0)
            pltpu.async_copy(table_hbm.at[fidx.at[p]],
                             rows.at[p], sem_g0 if p == 0 else sem_g1)

        def wait_gather(p):
            pltpu.make_async_copy(
                table_hbm.at[pl.ds(0, RPC)], rows.at[p],
                sem_g0 if p == 0 else sem_g1).wait()

        def out_slice(s0, tj, cl):
            return out_hbm.at[pl.ds(s0, SH), :, tj, :, pl.ds(cl, CB)]

        def wait_out(p):
            pltpu.make_async_copy(
                outb.at[p], out_slice(0, 0, b_base % 128),
                sem_o0 if p == 0 else sem_o1).wait()

        def transpose_and_fire(i, p):
            def s_body(s, carry):
                s_vec = jnp.full((L,), 0, jnp.int32) + s
                for b in range(0, CB, 2):
                    r0 = s * CB + b
                    v = [rows[p, r0, pl.ds(0, L)],
                         rows[p, r0, pl.ds(L, L)],
                         rows[p, r0 + 1, pl.ds(0, L)],
                         rows[p, r0 + 1, pl.ds(L, L)]]
                    b_vec = jnp.full((L,), b, jnp.int32)
                    b_vec1 = jnp.full((L,), b + 1, jnp.int32)
                    plsc.store_scatter(
                        outb.at[p], [s_vec, ti_lo, rl, b_vec], v[0])
                    plsc.store_scatter(
                        outb.at[p], [s_vec, ti_hi, rl, b_vec], v[1])
                    plsc.store_scatter(
                        outb.at[p], [s_vec, ti_lo, rl, b_vec1], v[2])
                    plsc.store_scatter(
                        outb.at[p], [s_vec, ti_hi, rl, b_vec1], v[3])
                return carry

            lax.fori_loop(0, SH, s_body, 0)
            s0, b0 = chunk_coords(i)
            pltpu.async_copy(
                outb.at[p],
                out_slice(s0, lax.div(b0, 128), lax.rem(b0, 128)),
                sem_o0 if p == 0 else sem_o1)

        stage_and_fire(0, 0)

        def body(i, carry):
            p = lax.rem(i, 2)

            @pl.when(i + 1 < n_chunks)
            def _():
                @pl.when(p == 0)
                def _():
                    stage_and_fire(i + 1, 1)

                @pl.when(p == 1)
                def _():
                    stage_and_fire(i + 1, 0)

            @pl.when(p == 0)
            def _():
                wait_gather(0)

                @pl.when(i >= 2)
                def _():
                    wait_out(0)

                transpose_and_fire(i, 0)

            @pl.when(p == 1)
            def _():
                wait_gather(1)

                @pl.when(i >= 2)
                def _():
                    wait_out(1)

                transpose_and_fire(i, 1)

            return carry

        lax.fori_loop(0, n_chunks, body, 0)
        wait_out(0)
        wait_out(1)

    return gather_kernel


def kernel(input, weight):
    idx_t = jnp.transpose(input).astype(jnp.int32)
    out5 = _make_gather(idx_t.shape[1], weight.shape[0])(idx_t, weight)
    BT = idx_t.shape[1]
    out = jnp.transpose(out5, (2, 4, 0, 1, 3))
    return jnp.reshape(out, (BT, S, D))
